# baseline (speedup 1.0000x reference)
import functools
import math

import jax
import jax.numpy as jnp
from jax import lax
from jax.experimental import pallas as pl
from jax.experimental.pallas import tpu as pltpu
from jax.experimental.pallas import tpu_sc as plsc

MAXLEN = 2048
D = 512
C = 256
DC = D + C
S = 1024
B = 16
NCORE = 2
NSUB = 16
CHP = 64
SS = 72
CHT = 128
ZR = 384
RW = 64


def _ts_body(o_ref):
    nt = C // 2
    log_inc = math.log(10000.0) / (nt - 1.0)
    j = lax.broadcasted_iota(jnp.int32, (MAXLEN, nt), 1).astype(jnp.float32)
    p = lax.broadcasted_iota(jnp.int32, (MAXLEN, nt), 0).astype(jnp.float32)
    st = p * jnp.exp(j * (-log_inc))
    o_ref[:, 0:nt] = jnp.sin(st)
    o_ref[:, nt:C] = jnp.cos(st)


def _prefix(rem, piece):
    return (rem // (2 * piece)) * (2 * piece)


def _sc_body(state_hbm, token_hbm, sl_hbm, tl_hbm, ts_hbm,
             out_hbm, len_hbm,
             buf0, buf1, sbuf, hbuf, slv, tlv, lnv, ts_sp, zero_sp,
             sin0, sin1, sout0, sout1, sts, sz):
    cid = lax.axis_index("c")
    sid = lax.axis_index("s")
    b = cid * (B // NCORE) + sid // 2
    h = sid % 2
    bufs = (buf0, buf1)
    sin = (sin0, sin1)
    sout = (sout0, sout1)

    zero16 = jnp.zeros((16,), jnp.float32)
    zr_t = ZR // NSUB

    def zrow(r_, _):
        for kk in range(D // 16):
            buf1[r_, pl.ds(kk * 16, 16)] = zero16
        return 0
    lax.fori_loop(0, zr_t, zrow, 0)
    pltpu.sync_copy(buf1.at[pl.ds(0, zr_t), :],
                    zero_sp.at[pl.ds(sid * zr_t, zr_t), pl.ds(0, D)])
    pltpu.sync_copy(buf1.at[pl.ds(0, zr_t), pl.ds(0, C)],
                    zero_sp.at[pl.ds(sid * zr_t, zr_t), pl.ds(D, C)])

    @pl.when(sid == 0)
    def _():
        pltpu.sync_copy(ts_hbm, ts_sp)

    pltpu.sync_copy(sl_hbm, slv)
    pltpu.sync_copy(tl_hbm, tlv)
    lanes = lax.broadcasted_iota(jnp.int32, (16,), 0)
    slvec = slv[...]
    tlvec = tlv[...]
    onb = lanes == b
    sl = jnp.max(jnp.where(onb, slvec, 0))
    tl = jnp.max(jnp.where(onb, tlvec, 0))
    ln = jnp.minimum(sl + tl, MAXLEN)
    F = pl.multiple_of((sl // 8) * 8, 8)
    r = sl - F
    L = ln - F
    Lc = pl.multiple_of(((L + 7) // 8) * 8, 8)
    G = pl.multiple_of((ln // 8) * 8, 8)
    E = pl.multiple_of(F + Lc, 8)

    @pl.when(jnp.logical_and(cid == 0, sid == 0))
    def _():
        lnv[...] = jnp.minimum(slvec + tlvec, MAXLEN)
        pltpu.sync_copy(lnv, len_hbm)

    plsc.subcore_barrier()

    def _ts_dma(start, rows):
        return pltpu.make_async_copy(
            ts_sp.at[pl.ds(start, rows), :],
            out_hbm.at[b, pl.ds(start, rows), pl.ds(D, C)], sts)

    nts = G // CHT
    for jj in range(MAXLEN // CHT):
        @pl.when(jnp.logical_and((jj % 2) == h, jj < nts))
        def _(jj=jj):
            _ts_dma(CHT * jj, CHT).start()
    tb = pl.multiple_of(CHT * nts, 8)
    tr = G - tb
    for piece in (64, 32, 16, 8):
        pc = jnp.logical_and(h == 0, (tr // piece) % 2 == 1)
        @pl.when(pc)
        def _(piece=piece):
            _ts_dma(pl.multiple_of(tb + _prefix(tr, piece), 8),
                    piece).start()

    rem = MAXLEN - E
    nz = rem // ZR
    r2 = rem - ZR * nz
    e64 = MAXLEN - ZR * nz
    n64 = r2 // RW
    r3 = r2 - RW * n64
    e8 = e64 - RW * n64
    n8 = r3 // 8

    def _zero_dma(start, rows):
        return pltpu.make_async_copy(
            zero_sp.at[pl.ds(0, rows), :],
            out_hbm.at[b, pl.ds(pl.multiple_of(start, 8), rows), :], sz)

    def _zero_all(fire):
        for j in range(MAXLEN // ZR):
            @pl.when(jnp.logical_and((j % 2) == h, j < nz))
            def _(j=j):
                dma = _zero_dma(MAXLEN - ZR * (j + 1), ZR)
                dma.start() if fire else dma.wait()
        for j in range(ZR // RW - 1):
            @pl.when(jnp.logical_and((j % 2) == h, j < n64))
            def _(j=j):
                dma = _zero_dma(e64 - RW * (j + 1), RW)
                dma.start() if fire else dma.wait()
        for j in range(RW // 8 - 1):
            @pl.when(jnp.logical_and((j % 2) == h, j < n8))
            def _(j=j):
                dma = _zero_dma(e8 - 8 * (j + 1), 8)
                dma.start() if fire else dma.wait()

    _zero_all(True)

    pltpu.sync_copy(state_hbm.at[b, pl.ds(F, 8), :], sbuf)

    def _stream_window(o, size):
        tst = jnp.clip(((o - r) // 8) * 8, 0, S - SS)
        shift = (o - r) - tst
        pltpu.sync_copy(
            token_hbm.at[b, pl.ds(pl.multiple_of(tst, 8), SS), :], buf0)
        head = jnp.where(o == 0, r, 0)
        fast = jnp.logical_and(head == 0, o + size <= L)

        @pl.when(fast)
        def _():
            def frow(iw, _):
                for kk in range(D // 16):
                    buf1[iw, pl.ds(kk * 16, 16)] = \
                        buf0[shift + iw, pl.ds(kk * 16, 16)]
                return 0
            lax.fori_loop(0, size, frow, 0)

        @pl.when(jnp.logical_not(fast))
        def _():
            def srow(iw, _):
                rr = jnp.clip(shift + iw, 0, SS - 1)
                use_state = iw < head
                valid = (o + iw) < L
                for kk in range(D // 16):
                    tv = buf0[rr, pl.ds(kk * 16, 16)]
                    sv = sbuf[jnp.minimum(iw, 7), pl.ds(kk * 16, 16)]
                    v = jnp.where(use_state, sv, tv)
                    buf1[iw, pl.ds(kk * 16, 16)] = jnp.where(valid, v, zero16)
                return 0
            lax.fori_loop(0, size, srow, 0)

        pltpu.sync_copy(
            buf1.at[pl.ds(0, size), :],
            out_hbm.at[b, pl.ds(pl.multiple_of(F + o, 8), size),
                       pl.ds(0, D)])

    nwin = Lc // CHP
    for w in range(S // CHP // 2):
        o = CHP * (2 * w) + CHP * h
        @pl.when(2 * w + h < nwin)
        def _(o=o):
            _stream_window(o, CHP)
    wb = pl.multiple_of(CHP * nwin, 8)
    wr = Lc - wb
    for piece in (32, 16, 8):
        pc = jnp.logical_and(h == 1, (wr // piece) % 2 == 1)
        @pl.when(pc)
        def _(piece=piece):
            _stream_window(wb + _prefix(wr, piece), piece)

    def srow_(i):
        return CHP * (2 * i) + CHP * h

    def cond(i):
        return srow_(i) + CHP <= F

    def cin(i):
        return pltpu.make_async_copy(
            state_hbm.at[b, pl.ds(pl.multiple_of(srow_(i), 8), CHP), :],
            bufs[i % 2].at[pl.ds(0, CHP), :], sin[i % 2])

    def cout(i):
        return pltpu.make_async_copy(
            bufs[i % 2].at[pl.ds(0, CHP), :],
            out_hbm.at[b, pl.ds(pl.multiple_of(srow_(i), 8), CHP),
                       pl.ds(0, D)], sout[i % 2])

    n = S // CHP // 2
    for i in range(n):
        if i >= 2:
            @pl.when(cond(i - 2))
            def _(i=i):
                cout(i - 2).wait()
        @pl.when(cond(i))
        def _(i=i):
            cin(i).start()
        if i >= 1:
            @pl.when(cond(i - 1))
            def _(i=i):
                cin(i - 1).wait()
                cout(i - 1).start()
    for i in (n - 2, n - 1):
        @pl.when(cond(i))
        def _(i=i):
            if i == n - 1:
                cin(i).wait()
                cout(i).start()
            cout(i).wait()

    ns8 = (F - (F // RW) * RW) // 8
    sb8 = pl.multiple_of((F // RW) * RW, 8)
    for g in range(RW // 8 - 1):
        @pl.when(jnp.logical_and(h == 0, g < ns8))
        def _(g=g):
            st = pl.multiple_of(sb8 + 8 * g, 8)
            pltpu.sync_copy(state_hbm.at[b, pl.ds(st, 8), :],
                            buf0.at[pl.ds(0, 8), :])
            pltpu.sync_copy(buf0.at[pl.ds(0, 8), :],
                            out_hbm.at[b, pl.ds(st, 8), pl.ds(0, D)])

    @pl.when(h == 1)
    def _():
        pltpu.sync_copy(ts_sp.at[pl.ds(G, 8), :], hbuf)

        def hz(g, _):
            for kk in range(C // 16):
                hbuf[g, pl.ds(kk * 16, 16)] = zero16
            return 0
        lax.fori_loop(ln - G, 8, hz, 0)
        pltpu.sync_copy(hbuf, out_hbm.at[b, pl.ds(G, 8), pl.ds(D, C)])

    _zero_all(False)
    for jj in range(MAXLEN // CHT):
        @pl.when(jnp.logical_and((jj % 2) == h, jj < nts))
        def _(jj=jj):
            _ts_dma(CHT * jj, CHT).wait()
    for piece in (64, 32, 16, 8):
        pc = jnp.logical_and(h == 0, (tr // piece) % 2 == 1)
        @pl.when(pc)
        def _(piece=piece):
            _ts_dma(pl.multiple_of(tb + _prefix(tr, piece), 8),
                    piece).wait()


@jax.jit
def kernel(state_seq, state_seq_length, token_seq, token_seq_length):
    sl = state_seq_length.astype(jnp.int32)
    tl = token_seq_length.astype(jnp.int32)
    ts = pl.pallas_call(
        _ts_body,
        out_shape=jax.ShapeDtypeStruct((MAXLEN, C), jnp.float32),
    )()
    sc = pl.kernel(
        _sc_body,
        out_type=(
            jax.ShapeDtypeStruct((B, MAXLEN, DC), jnp.float32),
            jax.ShapeDtypeStruct((B,), jnp.int32),
        ),
        mesh=plsc.VectorSubcoreMesh(core_axis_name="c", subcore_axis_name="s"),
        compiler_params=pltpu.CompilerParams(needs_layout_passes=False),
        scratch_types=[
            pltpu.VMEM((SS, D), jnp.float32),
            pltpu.VMEM((CHP, D), jnp.float32),
            pltpu.VMEM((8, D), jnp.float32),
            pltpu.VMEM((8, C), jnp.float32),
            pltpu.VMEM((B,), jnp.int32),
            pltpu.VMEM((B,), jnp.int32),
            pltpu.VMEM((B,), jnp.int32),
            pltpu.VMEM_SHARED((MAXLEN, C), jnp.float32),
            pltpu.VMEM_SHARED((ZR, DC), jnp.float32),
            pltpu.SemaphoreType.DMA,
            pltpu.SemaphoreType.DMA,
            pltpu.SemaphoreType.DMA,
            pltpu.SemaphoreType.DMA,
            pltpu.SemaphoreType.DMA,
            pltpu.SemaphoreType.DMA,
        ],
    )
    out, ln = sc(state_seq, token_seq, sl, tl, ts)
    return out, ln


# speedup vs baseline: 1.1746x; 1.1746x over previous
import functools
import math

import jax
import jax.numpy as jnp
from jax import lax
from jax.experimental import pallas as pl
from jax.experimental.pallas import tpu as pltpu
from jax.experimental.pallas import tpu_sc as plsc

MAXLEN = 2048
D = 512
C = 256
DC = D + C
S = 1024
SP = S + 8
B = 16
NCORE = 2
NSUB = 16
CHP = 64
CHT = 128
ZR = 512
RW = 64


def _ts_body(o_ref):
    nt = C // 2
    log_inc = math.log(10000.0) / (nt - 1.0)
    j = lax.broadcasted_iota(jnp.int32, (MAXLEN, nt), 1).astype(jnp.float32)
    p = lax.broadcasted_iota(jnp.int32, (MAXLEN, nt), 0).astype(jnp.float32)
    st = p * jnp.exp(j * (-log_inc))
    o_ref[:, 0:nt] = jnp.sin(st)
    o_ref[:, nt:C] = jnp.cos(st)


def _prestage_body(sl_ref, tl_ref, st8_ref, token_ref, ts8_ref,
                   tok2_ref, hts_ref):
    b = pl.program_id(0)
    sl = sl_ref[b]
    tl = tl_ref[b]
    ln = jnp.minimum(sl + tl, MAXLEN)
    r = sl - (sl // 8) * 8
    tokpad = jnp.concatenate(
        [token_ref[0], jnp.zeros((SP - S, D), jnp.float32)], axis=0)
    rolled = pltpu.roll(tokpad, r, 0)
    stpad = jnp.concatenate(
        [st8_ref[0], jnp.zeros((SP - 8, D), jnp.float32)], axis=0)
    i2 = lax.broadcasted_iota(jnp.int32, (SP, D), 0)
    main = jnp.where(i2 < r, stpad, rolled)
    tok2_ref[0] = jnp.where(i2 < r + tl, main, 0.0)
    i8 = lax.broadcasted_iota(jnp.int32, (8, C), 0)
    hts_ref[0] = jnp.where(i8 < ln - (ln // 8) * 8, ts8_ref[...], 0.0)


def _sc_body(state_hbm, tok2_hbm, sl_hbm, tl_hbm, ts_hbm, hts_hbm,
             out_hbm, len_hbm,
             buf0, buf1, sbuf, hbuf, slv, tlv, lnv, ts_sp, zero_sp,
             sin0, sin1, sout0, sout1, sts, sz):
    cid = lax.axis_index("c")
    sid = lax.axis_index("s")
    b = cid * (B // NCORE) + sid // 2
    h = sid % 2
    bufs = (buf0, buf1)
    sin = (sin0, sin1)
    sout = (sout0, sout1)

    zero16 = jnp.zeros((16,), jnp.float32)
    zr_t = ZR // NSUB

    def zrow(r_, _):
        for kk in range(D // 16):
            buf0[r_, pl.ds(kk * 16, 16)] = zero16
        return 0
    lax.fori_loop(0, zr_t, zrow, 0)
    pltpu.sync_copy(buf0.at[pl.ds(0, zr_t), :],
                    zero_sp.at[pl.ds(sid * zr_t, zr_t), pl.ds(0, D)])
    pltpu.sync_copy(buf0.at[pl.ds(0, zr_t), pl.ds(0, C)],
                    zero_sp.at[pl.ds(sid * zr_t, zr_t), pl.ds(D, C)])

    @pl.when(sid == 0)
    def _():
        pltpu.sync_copy(ts_hbm, ts_sp)

    pltpu.sync_copy(sl_hbm, slv)
    pltpu.sync_copy(tl_hbm, tlv)
    lanes = lax.broadcasted_iota(jnp.int32, (16,), 0)
    slvec = slv[...]
    tlvec = tlv[...]
    onb = lanes == b
    sl = jnp.max(jnp.where(onb, slvec, 0))
    tl = jnp.max(jnp.where(onb, tlvec, 0))
    ln = jnp.minimum(sl + tl, MAXLEN)
    F = pl.multiple_of((sl // 8) * 8, 8)
    Lc = pl.multiple_of(((ln - F + 7) // 8) * 8, 8)
    G = pl.multiple_of((ln // 8) * 8, 8)
    E = pl.multiple_of(F + Lc, 8)

    @pl.when(jnp.logical_and(cid == 0, sid == 0))
    def _():
        lnv[...] = jnp.minimum(slvec + tlvec, MAXLEN)
        pltpu.sync_copy(lnv, len_hbm)

    plsc.subcore_barrier()

    def _pipe(src_hbm, limit, dst_off):
        def srow(i):
            return CHP * (2 * i) + CHP * h

        def cond(i):
            return srow(i) + CHP <= limit

        def cin(i):
            return pltpu.make_async_copy(
                src_hbm.at[b, pl.ds(pl.multiple_of(srow(i), 8), CHP), :],
                bufs[i % 2], sin[i % 2])

        def cout(i):
            return pltpu.make_async_copy(
                bufs[i % 2],
                out_hbm.at[b, pl.ds(pl.multiple_of(dst_off + srow(i), 8),
                                    CHP), pl.ds(0, D)],
                sout[i % 2])

        n = S // CHP // 2
        for i in range(n):
            if i >= 2:
                @pl.when(cond(i - 2))
                def _(i=i):
                    cout(i - 2).wait()
            @pl.when(cond(i))
            def _(i=i):
                cin(i).start()
            if i >= 1:
                @pl.when(cond(i - 1))
                def _(i=i):
                    cin(i - 1).wait()
                    cout(i - 1).start()
        for i in (n - 2, n - 1):
            @pl.when(cond(i))
            def _(i=i):
                if i == n - 1:
                    cin(i).wait()
                    cout(i).start()
                cout(i).wait()

    def _ts_dma(start, rows):
        return pltpu.make_async_copy(
            ts_sp.at[pl.ds(start, rows), :],
            out_hbm.at[b, pl.ds(start, rows), pl.ds(D, C)], sts)

    nts = G // CHT
    for jj in range(MAXLEN // CHT):
        @pl.when(jnp.logical_and((jj % 2) == h, jj < nts))
        def _(jj=jj):
            _ts_dma(CHT * jj, CHT).start()
    tb = pl.multiple_of(CHT * nts, 8)
    tr = G - tb
    for piece in (64, 32, 16, 8):
        pc = jnp.logical_and(h == 0, (tr // piece) % 2 == 1)
        @pl.when(pc)
        def _(piece=piece):
            _ts_dma(pl.multiple_of(tb + _ts_prefix(tr, piece), 8),
                    piece).start()

    rem = MAXLEN - E
    n512 = rem // ZR
    r2 = rem - ZR * n512
    e64 = MAXLEN - ZR * n512
    n64 = r2 // RW
    r3 = r2 - RW * n64
    e8 = e64 - RW * n64
    n8 = r3 // 8

    def _zero_dma(start, rows):
        return pltpu.make_async_copy(
            zero_sp.at[pl.ds(0, rows), :],
            out_hbm.at[b, pl.ds(pl.multiple_of(start, 8), rows), :], sz)

    def _zero_all(fire):
        for j in range(MAXLEN // ZR):
            @pl.when(jnp.logical_and((j % 2) == h, j < n512))
            def _(j=j):
                dma = _zero_dma(MAXLEN - ZR * (j + 1), ZR)
                dma.start() if fire else dma.wait()
        for j in range(ZR // RW - 1):
            @pl.when(jnp.logical_and((j % 2) == h, j < n64))
            def _(j=j):
                dma = _zero_dma(e64 - RW * (j + 1), RW)
                dma.start() if fire else dma.wait()
        for j in range(RW // 8 - 1):
            @pl.when(jnp.logical_and((j % 2) == h, j < n8))
            def _(j=j):
                dma = _zero_dma(e8 - 8 * (j + 1), 8)
                dma.start() if fire else dma.wait()

    _zero_all(True)

    _pipe(state_hbm, F, 0)
    _pipe(tok2_hbm, Lc, F)

    ns8 = (F - (F // RW) * RW) // 8
    sb8 = pl.multiple_of((F // RW) * RW, 8)
    for g in range(RW // 8 - 1):
        @pl.when(jnp.logical_and(h == 0, g < ns8))
        def _(g=g):
            st = pl.multiple_of(sb8 + 8 * g, 8)
            pltpu.sync_copy(state_hbm.at[b, pl.ds(st, 8), :], sbuf)
            pltpu.sync_copy(sbuf, out_hbm.at[b, pl.ds(st, 8), pl.ds(0, D)])
    nk8 = (Lc - (Lc // RW) * RW) // 8
    kb8 = pl.multiple_of((Lc // RW) * RW, 8)
    for g in range(RW // 8 - 1):
        @pl.when(jnp.logical_and(h == 1, g < nk8))
        def _(g=g):
            st = pl.multiple_of(kb8 + 8 * g, 8)
            pltpu.sync_copy(tok2_hbm.at[b, pl.ds(st, 8), :], sbuf)
            pltpu.sync_copy(
                sbuf, out_hbm.at[b, pl.ds(pl.multiple_of(F + st, 8), 8),
                                 pl.ds(0, D)])
    @pl.when(h == 1)
    def _():
        pltpu.sync_copy(hts_hbm.at[b], hbuf)
        pltpu.sync_copy(hbuf, out_hbm.at[b, pl.ds(G, 8), pl.ds(D, C)])

    _zero_all(False)
    for jj in range(MAXLEN // CHT):
        @pl.when(jnp.logical_and((jj % 2) == h, jj < nts))
        def _(jj=jj):
            _ts_dma(CHT * jj, CHT).wait()
    for piece in (64, 32, 16, 8):
        pc = jnp.logical_and(h == 0, (tr // piece) % 2 == 1)
        @pl.when(pc)
        def _(piece=piece):
            _ts_dma(pl.multiple_of(tb + _ts_prefix(tr, piece), 8),
                    piece).wait()


def _ts_prefix(tr, piece):
    return (tr // (2 * piece)) * (2 * piece)


@jax.jit
def kernel(state_seq, state_seq_length, token_seq, token_seq_length):
    sl = state_seq_length.astype(jnp.int32)
    tl = token_seq_length.astype(jnp.int32)
    ts = pl.pallas_call(
        _ts_body,
        out_shape=jax.ShapeDtypeStruct((MAXLEN, C), jnp.float32),
    )()
    tok2, hts = pl.pallas_call(
        _prestage_body,
        grid_spec=pltpu.PrefetchScalarGridSpec(
            num_scalar_prefetch=2,
            grid=(B,),
            in_specs=[
                pl.BlockSpec((1, 8, D),
                             lambda b, slr, tlr: (b, slr[b] // 8, 0)),
                pl.BlockSpec((1, S, D), lambda b, slr, tlr: (b, 0, 0)),
                pl.BlockSpec(
                    (8, C),
                    lambda b, slr, tlr:
                    (jnp.minimum(slr[b] + tlr[b], MAXLEN) // 8, 0)),
            ],
            out_specs=[
                pl.BlockSpec((1, SP, D), lambda b, slr, tlr: (b, 0, 0)),
                pl.BlockSpec((1, 8, C), lambda b, slr, tlr: (b, 0, 0)),
            ],
        ),
        out_shape=[
            jax.ShapeDtypeStruct((B, SP, D), jnp.float32),
            jax.ShapeDtypeStruct((B, 8, C), jnp.float32),
        ],
    )(sl, tl, state_seq, token_seq, ts)
    sc = pl.kernel(
        _sc_body,
        out_type=(
            jax.ShapeDtypeStruct((B, MAXLEN, DC), jnp.float32),
            jax.ShapeDtypeStruct((B,), jnp.int32),
        ),
        mesh=plsc.VectorSubcoreMesh(core_axis_name="c", subcore_axis_name="s"),
        compiler_params=pltpu.CompilerParams(needs_layout_passes=False),
        scratch_types=[
            pltpu.VMEM((CHP, D), jnp.float32),
            pltpu.VMEM((CHP, D), jnp.float32),
            pltpu.VMEM((8, D), jnp.float32),
            pltpu.VMEM((8, C), jnp.float32),
            pltpu.VMEM((B,), jnp.int32),
            pltpu.VMEM((B,), jnp.int32),
            pltpu.VMEM((B,), jnp.int32),
            pltpu.VMEM_SHARED((MAXLEN, C), jnp.float32),
            pltpu.VMEM_SHARED((ZR, DC), jnp.float32),
            pltpu.SemaphoreType.DMA,
            pltpu.SemaphoreType.DMA,
            pltpu.SemaphoreType.DMA,
            pltpu.SemaphoreType.DMA,
            pltpu.SemaphoreType.DMA,
            pltpu.SemaphoreType.DMA,
        ],
    )
    out, ln = sc(state_seq, tok2, sl, tl, ts, hts)
    return out, ln
